# baseline (speedup 1.0000x reference)
import functools
import math

import jax
import jax.numpy as jnp
from jax.experimental import pallas as pl
from jax.experimental.pallas import tpu as pltpu

_D = 1024
_K = 32
_W = 256
_B = 256
_NRB = 8
_ALPHA = 0.9
_BB = 4


def _fused_body(gates_ref, trust_ref, vmr_ref, vmi_ref, cheby_ref, rrw_ref,
                riw_ref, rr_ref, ri_ref, cwt_ref, rwt_ref, x_ref, out_ref):
    gc = gates_ref[0]
    gi = gates_ref[1]
    cheby = cheby_ref[...]

    c0 = jnp.concatenate(
        [jnp.dot(cheby, x_ref[bb, :, :], preferred_element_type=jnp.float32)
         for bb in range(_BB)], axis=0)

    mixed = jnp.dot(c0, cwt_ref[...], preferred_element_type=jnp.float32)
    coeffs = c0 + gc * (mixed - c0)

    vr_rows = []
    vi_rows = []
    for bb in range(_BB):
        blk = coeffs[bb * _K:(bb + 1) * _K, :]
        vr_rows.append(jnp.sum(blk * rrw_ref[...], axis=0, keepdims=True))
        vi_rows.append(jnp.sum(blk * riw_ref[...], axis=0, keepdims=True))
    Vr = jnp.concatenate(vr_rows, axis=0) + vmr_ref[...]
    Vi = jnp.concatenate(vi_rows, axis=0) + vmi_ref[...]

    Sr = jnp.dot(trust_ref[...], rr_ref[...],
                 preferred_element_type=jnp.float32)
    Si = jnp.dot(trust_ref[...], ri_ref[...],
                 preferred_element_type=jnp.float32)
    u = Vr * Sr + Vi * Si

    retr = jnp.dot(u, rwt_ref[...], preferred_element_type=jnp.float32)

    for bb in range(_BB):
        out_ref[bb, :, :] = x_ref[bb, :, :] + gi * retr[bb:bb + 1, :]


@jax.jit
def kernel(x, w_bands, freq_bias, inject_gate, blend_gate, compress_gate,
           band_trust, compress_W, retrieve_W, roles_real, roles_imag,
           V_mem_real, V_mem_imag, cheby_mat, lanczos_sigma):
    f32 = jnp.float32
    gc = jax.nn.sigmoid(compress_gate)
    gi = jax.nn.sigmoid(inject_gate)[0] * blend_gate
    gates = jnp.stack([gc, gi]).astype(f32)

    ks = jnp.arange(_K)
    scale = jnp.where(ks == 0, 1.0 / _W, 2.0 / _W).astype(f32)
    band = (w_bands + freq_bias) * scale * lanczos_sigma * (1.0 - _ALPHA)
    rrw = band[:, None] * roles_real
    riw = band[:, None] * roles_imag
    trust = jnp.where(ks < _NRB, jax.nn.sigmoid(band_trust),
                      0.0).astype(f32)[None, :]
    vmr = (_ALPHA * V_mem_real)[None, :]
    vmi = (_ALPHA * V_mem_imag)[None, :]
    cwt = compress_W.T
    rwt = retrieve_W.T

    grid = (_B // _BB,)
    inv = lambda shape: pl.BlockSpec(shape, lambda i: (0,) * len(shape))
    out = pl.pallas_call(
        _fused_body,
        grid=grid,
        in_specs=[
            pl.BlockSpec(memory_space=pltpu.SMEM),
            inv((1, _K)),
            inv((1, _D)),
            inv((1, _D)),
            inv((_K, _W)),
            inv((_K, _D)),
            inv((_K, _D)),
            inv((_K, _D)),
            inv((_K, _D)),
            inv((_D, _D)),
            inv((_D, _D)),
            pl.BlockSpec((_BB, _W, _D), lambda i: (i, 0, 0)),
        ],
        out_specs=pl.BlockSpec((_BB, _W, _D), lambda i: (i, 0, 0)),
        out_shape=jax.ShapeDtypeStruct((_B, _W, _D), f32),
        compiler_params=pltpu.CompilerParams(
            dimension_semantics=(pltpu.PARALLEL,),
            vmem_limit_bytes=100 * 1024 * 1024,
        ),
    )(gates, trust, vmr, vmi, cheby_mat, rrw, riw, roles_real, roles_imag,
      cwt, rwt, x)
    return out


# speedup vs baseline: 1.0646x; 1.0646x over previous
import jax
import jax.numpy as jnp
from jax.experimental import pallas as pl
from jax.experimental.pallas import tpu as pltpu

_D = 1024
_K = 32
_W = 256
_B = 256
_NRB = 8
_ALPHA = 0.9
_BB = 8


def _fused_body(gates_ref, trust_ref, vmr_ref, vmi_ref, cheby_ref, rrw_ref,
                riw_ref, rr_ref, ri_ref, cwt_ref, rwt_ref, x_ref, out_ref):
    gc = gates_ref[0]
    gi = gates_ref[1]
    cheby = cheby_ref[...]

    c0 = jnp.concatenate(
        [jnp.dot(cheby, x_ref[bb, :, :].astype(jnp.bfloat16),
                 preferred_element_type=jnp.float32)
         for bb in range(_BB)], axis=0)

    mixed = jnp.dot(c0.astype(jnp.bfloat16), cwt_ref[...],
                    preferred_element_type=jnp.float32)
    coeffs = c0 + gc * (mixed - c0)

    vr_rows = []
    vi_rows = []
    for bb in range(_BB):
        blk = coeffs[bb * _K:(bb + 1) * _K, :]
        vr_rows.append(jnp.sum(blk * rrw_ref[...], axis=0, keepdims=True))
        vi_rows.append(jnp.sum(blk * riw_ref[...], axis=0, keepdims=True))
    Vr = jnp.concatenate(vr_rows, axis=0) + vmr_ref[...]
    Vi = jnp.concatenate(vi_rows, axis=0) + vmi_ref[...]

    Sr = jnp.dot(trust_ref[...], rr_ref[...],
                 preferred_element_type=jnp.float32)
    Si = jnp.dot(trust_ref[...], ri_ref[...],
                 preferred_element_type=jnp.float32)
    u = Vr * Sr + Vi * Si

    retr = jnp.dot(u.astype(jnp.bfloat16), rwt_ref[...],
                   preferred_element_type=jnp.float32)

    for bb in range(_BB):
        out_ref[bb, :, :] = x_ref[bb, :, :] + gi * retr[bb:bb + 1, :]


@jax.jit
def kernel(x, w_bands, freq_bias, inject_gate, blend_gate, compress_gate,
           band_trust, compress_W, retrieve_W, roles_real, roles_imag,
           V_mem_real, V_mem_imag, cheby_mat, lanczos_sigma):
    f32 = jnp.float32
    bf16 = jnp.bfloat16
    gc = jax.nn.sigmoid(compress_gate)
    gi = jax.nn.sigmoid(inject_gate)[0] * blend_gate
    gates = jnp.stack([gc, gi]).astype(f32)

    ks = jnp.arange(_K)
    scale = jnp.where(ks == 0, 1.0 / _W, 2.0 / _W).astype(f32)
    band = (w_bands + freq_bias) * scale * lanczos_sigma * (1.0 - _ALPHA)
    rrw = band[:, None] * roles_real
    riw = band[:, None] * roles_imag
    trust = jnp.where(ks < _NRB, jax.nn.sigmoid(band_trust),
                      0.0).astype(bf16)[None, :]
    vmr = (_ALPHA * V_mem_real)[None, :]
    vmi = (_ALPHA * V_mem_imag)[None, :]
    cwt = compress_W.T.astype(bf16)
    rwt = retrieve_W.T.astype(bf16)
    cheby_bf = cheby_mat.astype(bf16)
    rr_bf = roles_real.astype(bf16)
    ri_bf = roles_imag.astype(bf16)

    grid = (_B // _BB,)
    inv = lambda shape: pl.BlockSpec(shape, lambda i: (0,) * len(shape))
    out = pl.pallas_call(
        _fused_body,
        grid=grid,
        in_specs=[
            pl.BlockSpec(memory_space=pltpu.SMEM),
            inv((1, _K)),
            inv((1, _D)),
            inv((1, _D)),
            inv((_K, _W)),
            inv((_K, _D)),
            inv((_K, _D)),
            inv((_K, _D)),
            inv((_K, _D)),
            inv((_D, _D)),
            inv((_D, _D)),
            pl.BlockSpec((_BB, _W, _D), lambda i: (i, 0, 0)),
        ],
        out_specs=pl.BlockSpec((_BB, _W, _D), lambda i: (i, 0, 0)),
        out_shape=jax.ShapeDtypeStruct((_B, _W, _D), f32),
        compiler_params=pltpu.CompilerParams(
            dimension_semantics=(pltpu.PARALLEL,),
            vmem_limit_bytes=100 * 1024 * 1024,
        ),
    )(gates, trust, vmr, vmi, cheby_bf, rrw, riw, rr_bf, ri_bf,
      cwt, rwt, x)
    return out
